# baseline (speedup 1.0000x reference)
import functools
import math

import jax
import jax.numpy as jnp
from jax import lax
from jax.experimental import pallas as pl
from jax.experimental.pallas import tpu as pltpu
from jax.experimental.pallas import tpu_sc as plsc

_NC = 2
_NS = 16
_NW = _NC * _NS
_LANES = 16

_VOCAB = 1000000
_EMB = 64
_SCALE = math.sqrt(_EMB)

_CHUNK = 128
_NBUF = 8


def _body(tok_hbm, table_hbm, out_hbm, idx_v, rows, gsems, osems,
          *, n_chunks, n_steps):
    wid = lax.axis_index("s") * _NC + lax.axis_index("c")
    chunk0 = wid * n_chunks
    out_base = wid * n_chunks * _CHUNK

    pltpu.sync_copy(tok_hbm.at[pl.ds(chunk0, n_chunks)], idx_v)

    def gather(b, j):
        return pltpu.make_async_copy(
            table_hbm.at[idx_v.at[j]], rows.at[b], gsems[b])

    def out_copy(b, j):
        return pltpu.make_async_copy(
            rows.at[b], out_hbm.at[pl.ds(out_base + j * _CHUNK, _CHUNK)],
            osems[b])

    def scale(b):
        @plsc.parallel_loop(0, _CHUNK, 1, unroll=4)
        def _(r):
            for c in range(_EMB // _LANES):
                sl = pl.ds(c * _LANES, _LANES)
                rows[b, r, sl] = rows[b, r, sl] * _SCALE

    for b in range(_NBUF):
        gather(b, b).start()

    @pl.loop(0, n_steps)
    def _(s):
        jb = s * _NBUF
        for b in range(_NBUF):
            @pl.when(s > 0)
            def _():
                out_copy(b, jb - _NBUF + b).wait()
                gather(b, jb + b).start()
        for b in range(_NBUF):
            gather(b, jb + b).wait()
            scale(b)
            out_copy(b, jb + b).start()

    for b in range(_NBUF):
        out_copy(b, (n_steps - 1) * _NBUF + b).wait()


def kernel(tokens, table):
    bsz, seq = tokens.shape
    vocab, emb = table.shape
    assert emb == _EMB
    n_tok = bsz * seq
    assert n_tok % (_NW * _CHUNK * _NBUF) == 0
    n_chunks = n_tok // (_NW * _CHUNK)
    n_steps = n_chunks // _NBUF

    tok2d = tokens.reshape(n_tok // _CHUNK, _CHUNK).astype(jnp.int32)
    table = table.astype(jnp.float32)

    mesh = plsc.VectorSubcoreMesh(
        core_axis_name="c", subcore_axis_name="s",
        num_cores=_NC, num_subcores=_NS)

    body = functools.partial(_body, n_chunks=n_chunks, n_steps=n_steps)
    out = pl.kernel(
        body,
        out_type=jax.ShapeDtypeStruct((n_tok, _EMB), jnp.float32),
        mesh=mesh,
        compiler_params=pltpu.CompilerParams(use_tc_tiling_on_sc=False),
        scratch_types=dict(
            idx_v=pltpu.VMEM((n_chunks, _CHUNK), jnp.int32),
            rows=pltpu.VMEM((_NBUF, _CHUNK, _EMB), jnp.float32),
            gsems=[pltpu.SemaphoreType.DMA] * _NBUF,
            osems=[pltpu.SemaphoreType.DMA] * _NBUF,
        ),
    )(tok2d, table)
    return out.reshape(bsz, seq, emb)
